# Initial kernel scaffold; baseline (speedup 1.0000x reference)
import jax
import jax.numpy as jnp
from jax.experimental import pallas as pl


def kernel(logits, top_k):
    raise NotImplementedError("write your pallas kernel here")


# speedup vs baseline: 15.8895x; 15.8895x over previous
import jax
import jax.numpy as jnp
from jax.experimental import pallas as pl

_TOP_K = 50
_NEG_BIG = -1e10


def _row_kernel(x_ref, g_ref, probs_ref, tok_ref):
    x = x_ref[0]
    g = g_ref[0]

    xb = jax.lax.bitcast_convert_type(x, jnp.uint32)
    sign = xb >> jnp.uint32(31)
    mono = xb ^ (jnp.uint32(0x80000000) | (jnp.uint32(0) - sign))

    k_req = jnp.int32(_TOP_K)

    def body(i, kcur):
        bit = jnp.uint32(1) << (jnp.uint32(31) - i.astype(jnp.uint32))
        cand = kcur | bit
        cnt = jnp.sum((mono >= cand).astype(jnp.int32))
        return jnp.where(cnt >= k_req, cand, kcur)

    kbits = jax.lax.fori_loop(0, 32, body, jnp.uint32(0))

    keep = mono >= kbits
    m = jnp.max(x)
    e = jnp.where(keep, jnp.exp(x - m), jnp.float32(0.0))
    z = jnp.sum(e)
    probs_ref[0] = e / z

    y = jnp.where(keep, x + g, jnp.float32(_NEG_BIG))
    ymax = jnp.max(y)
    r, c = y.shape
    flat = (jax.lax.broadcasted_iota(jnp.int32, (r, c), 0) * c
            + jax.lax.broadcasted_iota(jnp.int32, (r, c), 1))
    tok = jnp.min(jnp.where(y == ymax, flat, jnp.int32(2**31 - 1)))
    tok_ref[0] = jnp.full(tok_ref.shape[1:], tok, jnp.int32)


def kernel(logits, top_k):
    del top_k
    b, v = logits.shape
    c = 1000
    assert v % c == 0
    r = v // c
    g = jax.random.gumbel(jax.random.key(1), (b, v), jnp.float32)
    xr = logits.reshape(b, r, c)
    gr = g.reshape(b, r, c)

    probs3, tok3 = pl.pallas_call(
        _row_kernel,
        grid=(b,),
        in_specs=[
            pl.BlockSpec((1, r, c), lambda i: (i, 0, 0)),
            pl.BlockSpec((1, r, c), lambda i: (i, 0, 0)),
        ],
        out_specs=[
            pl.BlockSpec((1, r, c), lambda i: (i, 0, 0)),
            pl.BlockSpec((1, 8, 128), lambda i: (i, 0, 0)),
        ],
        out_shape=[
            jax.ShapeDtypeStruct((b, r, c), jnp.float32),
            jax.ShapeDtypeStruct((b, 8, 128), jnp.int32),
        ],
    )(xr, gr)

    return probs3.reshape(b, v), tok3[:, 0, 0]


# speedup vs baseline: 41.1757x; 2.5914x over previous
import jax
import jax.numpy as jnp
from jax import lax
from jax.experimental import pallas as pl
from jax.experimental.pallas import tpu as pltpu
from jax.experimental.pallas import tpu_sc as plsc

_TOP_K = 50
_NEG = -1e30
_LANE = 128
_CHUNK_TILES = 13
_CHUNK = _CHUNK_TILES * _LANE
_GRP_V = 26
_NGRP = 4
_ROWB = 1040
_TRIG = 592
_OUTW = 64
_OVROW = 80
_NQ = 8


def _lane16():
    return lax.iota(jnp.int32, 16)


def _negv():
    return jnp.full((16,), _NEG, jnp.float32)


def _splat_u32(x):
    return jnp.full((16,), x, jnp.uint32)


def _unmono16(k_u32):
    top = k_u32 >= _splat_u32(0x80000000)
    bits = jnp.where(top, k_u32 ^ _splat_u32(0x80000000),
                     k_u32 ^ _splat_u32(0xFFFFFFFF))
    return plsc.bitcast(bits, jnp.float32)


def _sc_body_factory(b, v):
    tcols = (v + _LANE - 1) // _LANE
    nch = tcols // _CHUNK_TILES
    nfull = (nch // _NQ) * _NQ
    tpw = nfull // _NQ
    npair = (tpw + 1) // 2

    def body(x_hbm, vals_hbm, idx_hbm,
             chunk, bv, bi, cv, ci_s, ov, oi, sxf, sxi, cnt_s, tsp_s, sem0, sem1):
        w = lax.axis_index("s") * 2 + lax.axis_index("c")
        g = w // _NQ
        q = w % _NQ
        row0 = pl.multiple_of(g * 8, 8)
        lane = _lane16()
        negv = _negv()

        def popc(m):
            return plsc.all_reduce_population_count(m)

        def to_i32(vec):
            return vec[0]

        def to_f32(vec):
            return vec[0]


        def count_ge(rb, nvec, cand_f):
            def cb(j, acc):
                vv = bv[pl.ds(rb + j * 16, 16)]
                return acc + popc(vv >= cand_f)
            accv = lax.fori_loop(0, nvec, cb, jnp.zeros((16,), jnp.int32))
            return to_i32(accv)

        def buf_kth(rb, cnt):
            nvec = (cnt + 15) // 16

            def bit_body(i, k_cur):
                shift = _splat_u32(31) - jnp.full((16,), i, jnp.uint32)
                cand = k_cur | (_splat_u32(1) << shift)
                c = count_ge(rb, nvec, _unmono16(cand))
                return jnp.where(c >= _TOP_K, cand, k_cur)

            return _unmono16(lax.fori_loop(0, 32, bit_body, _splat_u32(0)))

        def compact(r, kth_f):
            rb = r * _ROWB
            cnt = cnt_s[r]
            nvec = (cnt + 15) // 16

            def cb(j, acc):
                vv = bv[pl.ds(rb + j * 16, 16)]
                ii = bi[pl.ds(rb + j * 16, 16)]
                m = vv >= kth_f
                n0 = to_i32(popc(m))
                plsc.store_compressed(cv.at[pl.ds(acc, 16)], vv, mask=m)
                plsc.store_compressed(ci_s.at[pl.ds(acc, 16)], ii, mask=m)
                return acc + n0

            ncnt = lax.fori_loop(0, nvec, cb, jnp.int32(0))

            def copy_back(j, _):
                bv[pl.ds(rb + j * 16, 16)] = cv[pl.ds(j * 16, 16)]
                bi[pl.ds(rb + j * 16, 16)] = ci_s[pl.ds(j * 16, 16)]
                return _

            nv2 = (ncnt + 15) // 16
            lax.fori_loop(0, nv2, copy_back, jnp.int32(0))
            j0 = ncnt // 16
            vv = bv[pl.ds(rb + j0 * 16, 16)]
            pos = j0 * 16 + lane
            bv[pl.ds(rb + j0 * 16, 16)] = jnp.where(pos < ncnt, vv, negv)

            def clear(j, _):
                bv[pl.ds(rb + j * 16, 16)] = negv
                return _

            lax.fori_loop(j0 + 1, nvec, clear, jnp.int32(0))
            cnt_s[r] = ncnt


        def process(par, ci, masked):
            limit = v - ci * _CHUNK

            def row_body(r, _):
                tsp = jnp.full((16,), tsp_s[r], jnp.float32)

                def grp_body(g4, _g):
                    base = g4 * (_GRP_V * 16)

                    def ld(k):
                        return chunk[par, r, pl.ds(base + k * 16, 16)]

                    def msk(k, vv):
                        if not masked:
                            return vv
                        pos = base + k * 16 + lane
                        return jnp.where(pos < limit, vv, negv)

                    gm = msk(0, ld(0))
                    for k in range(1, _GRP_V):
                        gm = jnp.maximum(gm, msk(k, ld(k)))

                    @pl.when(to_i32(popc(gm >= tsp)) > 0)
                    def _app():
                        @pl.when(cnt_s[r] >= _TRIG)
                        def _rc():
                            kth = buf_kth(r * _ROWB, cnt_s[r])
                            compact(r, kth)
                            tsp_s[r] = to_f32(kth)

                        tsp2 = jnp.full((16,), tsp_s[r], jnp.float32)
                        rb = r * _ROWB
                        cnt = cnt_s[r]
                        for k in range(_GRP_V):
                            vv = msk(k, ld(k))
                            m = vv >= tsp2
                            n0 = to_i32(popc(m))
                            iv = ci * _CHUNK + base + k * 16 + lane
                            plsc.store_compressed(
                                bv.at[pl.ds(rb + cnt, 16)], vv, mask=m)
                            plsc.store_compressed(
                                bi.at[pl.ds(rb + cnt, 16)], iv, mask=m)
                            cnt = cnt + n0
                        cnt_s[r] = cnt

                    return _g

                lax.fori_loop(0, _NGRP, grp_body, jnp.int32(0))
                return _

            lax.fori_loop(0, 8, row_body, jnp.int32(0))

        def init_buf(j, _):
            bv[pl.ds(j * 16, 16)] = negv
            return _

        lax.fori_loop(0, 8 * _ROWB // 16, init_buf, jnp.int32(0))

        def init_state(r, _):
            cnt_s[r] = jnp.int32(0)
            tsp_s[r] = jnp.float32(_NEG)
            return _

        lax.fori_loop(0, 8, init_state, jnp.int32(0))

        def dsc(s, par_ref, sem):
            ci = q + _NQ * s
            col0 = pl.multiple_of(ci * _CHUNK, _LANE)
            return pltpu.make_async_copy(
                x_hbm.at[pl.ds(row0, 8), pl.ds(col0, _CHUNK)],
                chunk.at[par_ref], sem)

        dsc(0, 0, sem0).start()
        dsc(1, 1, sem1).start()

        def pair(i, carry):
            s0 = 2 * i
            dsc(s0, 0, sem0).wait()
            process(0, q + _NQ * s0, masked=False)

            @pl.when(s0 + 2 < tpw)
            def _pf0():
                dsc(s0 + 2, 0, sem0).start()

            @pl.when(s0 + 1 < tpw)
            def _odd():
                dsc(s0 + 1, 1, sem1).wait()
                process(1, q + _NQ * (s0 + 1), masked=False)

                @pl.when(s0 + 3 < tpw)
                def _pf1():
                    dsc(s0 + 3, 1, sem1).start()

            return carry

        lax.fori_loop(0, npair, pair, jnp.int32(0))

        for extra in range(nch - nfull):
            @pl.when(q == extra)
            def _tail():
                ci_t = nfull + extra
                col0 = pl.multiple_of(ci_t * _CHUNK, _LANE)
                cp = pltpu.make_async_copy(
                    x_hbm.at[pl.ds(row0, 8), pl.ds(col0, _CHUNK)],
                    chunk.at[0], sem0)
                cp.start()
                cp.wait()
                process(0, ci_t, masked=True)

        def out_row(r, _):
            rb = r * _ROWB
            ob = r * _OVROW
            kth = buf_kth(rb, cnt_s[r])

            def init_out(j, _o):
                ov[pl.ds(ob + j * 16, 16)] = negv
                oi[pl.ds(ob + j * 16, 16)] = jnp.zeros((16,), jnp.int32)
                return _o

            lax.fori_loop(0, _OVROW // 16, init_out, jnp.int32(0))

            nvec = (cnt_s[r] + 15) // 16

            def ex(j, acc):
                vv = bv[pl.ds(rb + j * 16, 16)]
                ii = bi[pl.ds(rb + j * 16, 16)]
                m = vv >= kth
                n0 = to_i32(popc(m))

                def do(a):
                    plsc.store_compressed(ov.at[pl.ds(ob + a, 16)], vv, mask=m)
                    plsc.store_compressed(oi.at[pl.ds(ob + a, 16)], ii, mask=m)
                    return a + n0

                return lax.cond(acc <= _OUTW - 1, do, lambda a: a, acc)

            lax.fori_loop(0, nvec, ex, jnp.int32(0))

            off = (g * 8 + r) * (_NQ * _OUTW) + q * _OUTW
            pltpu.sync_copy(ov.at[pl.ds(ob, _OUTW)], vals_hbm.at[pl.ds(off, _OUTW)])
            pltpu.sync_copy(oi.at[pl.ds(ob, _OUTW)], idx_hbm.at[pl.ds(off, _OUTW)])
            return _

        lax.fori_loop(0, 8, out_row, jnp.int32(0))

    return body


def _sc_topk(x):
    b, v = x.shape
    chunk_len = _CHUNK
    mesh = plsc.VectorSubcoreMesh(core_axis_name="c", subcore_axis_name="s",
                                  num_cores=2, num_subcores=16)
    f = pl.kernel(
        _sc_body_factory(b, v),
        out_type=[
            jax.ShapeDtypeStruct((b * _NQ * _OUTW,), jnp.float32),
            jax.ShapeDtypeStruct((b * _NQ * _OUTW,), jnp.int32),
        ],
        mesh=mesh,
        compiler_params=pltpu.CompilerParams(needs_layout_passes=False),
        scratch_types=[
            pltpu.VMEM((2, 8, chunk_len), jnp.float32),
            pltpu.VMEM((8 * _ROWB,), jnp.float32),
            pltpu.VMEM((8 * _ROWB,), jnp.int32),
            pltpu.VMEM((_ROWB,), jnp.float32),
            pltpu.VMEM((_ROWB,), jnp.int32),
            pltpu.VMEM((8 * _OVROW,), jnp.float32),
            pltpu.VMEM((8 * _OVROW,), jnp.int32),
            pltpu.VMEM((16,), jnp.float32),
            pltpu.VMEM((16,), jnp.int32),
            pltpu.SMEM((8,), jnp.int32),
            pltpu.SMEM((8,), jnp.float32),
            pltpu.SemaphoreType.DMA,
            pltpu.SemaphoreType.DMA,
        ],
    )
    return f(x)


def _rotl(x, d):
    return (x << jnp.uint32(d)) | (x >> jnp.uint32(32 - d))


def _tf_rounds(x0, x1, rots):
    for r in rots:
        x0 = x0 + x1
        x1 = _rotl(x1, r)
        x1 = x0 ^ x1
    return x0, x1


def _gumbel_at(flat_idx_u32, k0, k1):
    u32 = jnp.uint32
    x0 = jnp.zeros_like(flat_idx_u32)
    x1 = flat_idx_u32
    ks0, ks1 = u32(k0), u32(k1)
    ks2 = ks0 ^ ks1 ^ u32(0x1BD11BDA)
    x0 = x0 + ks0
    x1 = x1 + ks1
    x0, x1 = _tf_rounds(x0, x1, (13, 15, 26, 6))
    x0, x1 = x0 + ks1, x1 + ks2 + u32(1)
    x0, x1 = _tf_rounds(x0, x1, (17, 29, 16, 24))
    x0, x1 = x0 + ks2, x1 + ks0 + u32(2)
    x0, x1 = _tf_rounds(x0, x1, (13, 15, 26, 6))
    x0, x1 = x0 + ks0, x1 + ks1 + u32(3)
    x0, x1 = _tf_rounds(x0, x1, (17, 29, 16, 24))
    x0, x1 = x0 + ks1, x1 + ks2 + u32(4)
    x0, x1 = _tf_rounds(x0, x1, (13, 15, 26, 6))
    x0, x1 = x0 + ks2, x1 + ks0 + u32(5)
    bits = x0 ^ x1
    float_bits = (bits >> u32(9)) | u32(0x3F800000)
    f = lax.bitcast_convert_type(float_bits, jnp.float32) - jnp.float32(1.0)
    tiny = jnp.float32(jnp.finfo(jnp.float32).tiny)
    u = jnp.maximum(tiny, f * (jnp.float32(1.0) - tiny) + tiny)
    return -jnp.log(-jnp.log(u))


def _sel_body(vals_ref, scores_ref, idx_ref, kth_ref, m_ref, zi_ref, tok_ref):
    vals = vals_ref[...]
    xb = lax.bitcast_convert_type(vals, jnp.uint32)
    sign = xb >> jnp.uint32(31)
    mono = xb ^ (jnp.uint32(0x80000000) | (jnp.uint32(0) - sign))

    def bit_body(i, k_cur):
        cand = k_cur | (jnp.uint32(1) << (jnp.uint32(31) - i.astype(jnp.uint32)))
        cnt = jnp.sum((mono >= cand).astype(jnp.int32), axis=1, keepdims=True)
        return jnp.where(cnt >= _TOP_K, cand, k_cur)

    kbits = lax.fori_loop(0, 32, bit_body,
                          jnp.zeros((vals.shape[0], 1), jnp.uint32))
    top = kbits >= jnp.uint32(0x80000000)
    kth = lax.bitcast_convert_type(
        jnp.where(top, kbits ^ jnp.uint32(0x80000000),
                  kbits ^ jnp.uint32(0xFFFFFFFF)), jnp.float32)

    keep = vals >= kth
    m = jnp.max(vals, axis=1, keepdims=True)
    e = jnp.where(keep, jnp.exp(vals - m), jnp.float32(0.0))
    z = jnp.sum(e, axis=1, keepdims=True)

    y = jnp.where(keep, scores_ref[...], jnp.float32(_NEG))
    ym = jnp.max(y, axis=1, keepdims=True)
    tok = jnp.min(jnp.where(y == ym, idx_ref[...], jnp.int32(2**31 - 1)),
                  axis=1, keepdims=True)

    shp = kth_ref.shape
    kth_ref[...] = jnp.broadcast_to(kth, shp)
    m_ref[...] = jnp.broadcast_to(m, shp)
    zi_ref[...] = jnp.broadcast_to(jnp.float32(1.0) / z, shp)
    tok_ref[...] = jnp.broadcast_to(tok, shp)


def _probs_body(kth_ref, m_ref, zi_ref, x_ref, probs_ref):
    x = x_ref[...]
    kth = kth_ref[:, 0:1]
    m = m_ref[:, 0:1]
    zi = zi_ref[:, 0:1]
    probs_ref[...] = jnp.where(x >= kth, jnp.exp(x - m) * zi, jnp.float32(0.0))


def kernel(logits, top_k):
    del top_k
    b, v = logits.shape
    nw = _NQ * _OUTW

    vals_f, idx_f = _sc_topk(logits)
    vals = vals_f.reshape(b, nw)
    idx = idx_f.reshape(b, nw)

    valid = vals > jnp.float32(-1e29)
    flat = (jnp.arange(b, dtype=jnp.uint32)[:, None] * jnp.uint32(v)
            + idx.astype(jnp.uint32))
    g = _gumbel_at(flat, 0, 1)
    scores = jnp.where(valid, vals + g, jnp.float32(_NEG))

    full = pl.BlockSpec((b, 128), lambda *_: (0, 0))
    wide = pl.BlockSpec((b, nw), lambda *_: (0, 0))
    kth128, m128, zi128, tok = pl.pallas_call(
        _sel_body,
        grid=(1,),
        in_specs=[wide, wide, wide],
        out_specs=[full, full, full, full],
        out_shape=[
            jax.ShapeDtypeStruct((b, 128), jnp.float32),
            jax.ShapeDtypeStruct((b, 128), jnp.float32),
            jax.ShapeDtypeStruct((b, 128), jnp.float32),
            jax.ShapeDtypeStruct((b, 128), jnp.int32),
        ],
    )(vals, scores, idx)

    cb = 8192
    grid = (v + cb - 1) // cb
    probs = pl.pallas_call(
        _probs_body,
        grid=(grid,),
        in_specs=[full, full, full, pl.BlockSpec((b, cb), lambda j: (0, j))],
        out_specs=pl.BlockSpec((b, cb), lambda j: (0, j)),
        out_shape=jax.ShapeDtypeStruct((b, v), jnp.float32),
    )(kth128, m128, zi128, logits)

    return probs, tok[:, 0]
